# baseline (speedup 1.0000x reference)
import functools

import jax
import jax.numpy as jnp
from jax import lax
from jax.experimental import pallas as pl
from jax.experimental.pallas import tpu as pltpu
from jax.experimental.pallas import tpu_sc as plsc

N = 10000
D = 128
NC = 2
NS = 16
NW = NC * NS
C = 128

N_PAD = 10112
ROWS_PER_TILE = N_PAD // NS
DEG_PAD = 10240
DEG_PER_TILE = DEG_PAD // NS


def _sc_segment_sum(x_pad, src_chunks, dst_chunks, zeros2d, zeros1d, cpt):
    mesh = plsc.VectorSubcoreMesh(core_axis_name="c", subcore_axis_name="s")

    npc = cpt // 2

    @functools.partial(
        pl.kernel,
        out_type=(
            jax.ShapeDtypeStruct((NC, N_PAD, D), jnp.float32),
            jax.ShapeDtypeStruct((NC, 1, DEG_PAD), jnp.float32),
        ),
        mesh=mesh,
        scratch_types=[
            pltpu.VMEM_SHARED((N_PAD, D), jnp.float32),
            pltpu.VMEM_SHARED((DEG_PAD,), jnp.float32),
            pltpu.VMEM((npc, C), jnp.int32),
            pltpu.VMEM((npc, C), jnp.int32),
            pltpu.VMEM((C, D), jnp.float32),
            pltpu.VMEM((C, D), jnp.float32),
            pltpu.VMEM((C,), jnp.float32),
        ] + [pltpu.SemaphoreType.DMA] * 6,
    )
    def kern(x_hbm, src_hbm, dst_hbm, z2_hbm, z1_hbm, out_sum, out_deg,
             acc_sh, deg_sh, src_v, dst_v, r0, r1, ones_v,
             g0, g1, s0, s1, d0, d1):
        rows = [r0, r1]
        gsems = [g0, g1]
        ssems = [s0, s1]
        dsems = [d0, d1]
        cid = lax.axis_index("c")
        sid = lax.axis_index("s")
        wid = cid * NS + sid

        def fire_gather(j, b):
            pltpu.async_copy(x_hbm.at[src_v.at[j]], rows[b], gsems[b])

        def wait_gather(j, b):
            pltpu.make_async_copy(x_hbm.at[src_v.at[j]], rows[b], gsems[b]).wait()

        def fire_scat(j, b):
            pltpu.async_copy(rows[b], acc_sh.at[dst_v.at[j]], ssems[b], add=True)
            pltpu.async_copy(ones_v, deg_sh.at[dst_v.at[j]], dsems[b], add=True)

        def wait_scat(j, b):
            pltpu.make_async_copy(rows[b], acc_sh.at[dst_v.at[j]], ssems[b]).wait()

        def wait_deg(j, b):
            pltpu.make_async_copy(ones_v, deg_sh.at[dst_v.at[j]], dsems[b]).wait()

        def visit(j, b, wait_prev_scat, fire_g, wait_prev_deg):
            o = 1 - b
            if wait_prev_scat:
                wait_scat(j - 1, o)
            if fire_g:
                fire_gather(j + 1, o)
            if wait_prev_deg:
                wait_deg(j - 2, b)
            wait_gather(j, b)
            fire_scat(j, b)

        pltpu.sync_copy(src_hbm.at[pl.ds(wid * cpt, npc)], src_v)
        pltpu.sync_copy(dst_hbm.at[pl.ds(wid * cpt, npc)], dst_v)
        for i in range(C // 16):
            ones_v[pl.ds(i * 16, 16)] = jnp.full((16,), 1.0, jnp.float32)
        fire_gather(0, 0)
        fire_gather(1, 1)

        zbase = sid * ROWS_PER_TILE
        for r in range(0, ROWS_PER_TILE, C):
            nrows = min(C, ROWS_PER_TILE - r)
            pltpu.sync_copy(z2_hbm.at[pl.ds(0, nrows)],
                            acc_sh.at[pl.ds(zbase + r, nrows)])
        pltpu.sync_copy(z1_hbm, deg_sh.at[pl.ds(sid * DEG_PER_TILE, DEG_PER_TILE)])

        plsc.subcore_barrier()

        for h in range(cpt // npc):
            if h > 0:
                pltpu.sync_copy(src_hbm.at[pl.ds(wid * cpt + h * npc, npc)],
                                src_v)
                pltpu.sync_copy(dst_hbm.at[pl.ds(wid * cpt + h * npc, npc)],
                                dst_v)
                fire_gather(0, 0)
                fire_gather(1, 1)
            visit(0, 0, False, False, False)
            visit(1, 1, True, True, False)

            def body(t, _):
                visit(2 * t, 0, True, True, True)
                visit(2 * t + 1, 1, True, True, True)
                return 0

            lax.fori_loop(1, npc // 2 - 1, body, 0)

            visit(npc - 2, 0, True, True, True)
            visit(npc - 1, 1, True, False, True)
            wait_scat(npc - 1, 1)
            wait_deg(npc - 2, 0)
            wait_deg(npc - 1, 1)

        plsc.subcore_barrier()

        pltpu.sync_copy(acc_sh.at[pl.ds(zbase, ROWS_PER_TILE)],
                        out_sum.at[cid, pl.ds(zbase, ROWS_PER_TILE)])
        pltpu.sync_copy(deg_sh.at[pl.ds(sid * DEG_PER_TILE, DEG_PER_TILE)],
                        out_deg.at[cid, 0,
                                   pl.ds(sid * DEG_PER_TILE, DEG_PER_TILE)])

    return kern(x_pad, src_chunks, dst_chunks, zeros2d, zeros1d)


RPK_KB = 128
RPK_RE = 125


def _repack(ei, dummy_src, dummy_dst, k_chunks):
    eb = RPK_RE * C

    def body(e_ref, dsrc_ref, ddst_ref, osrc_ref, odst_ref):
        osrc_ref[:RPK_RE] = e_ref[0].reshape(RPK_RE, C)
        odst_ref[:RPK_RE] = e_ref[1].reshape(RPK_RE, C)
        osrc_ref[RPK_RE:] = dsrc_ref[...]
        odst_ref[RPK_RE:] = ddst_ref[...]

    return pl.pallas_call(
        body,
        grid=(k_chunks // RPK_KB,),
        in_specs=[
            pl.BlockSpec((2, eb), lambda i: (0, i)),
            pl.BlockSpec((RPK_KB - RPK_RE, C), lambda i: (0, 0)),
            pl.BlockSpec((RPK_KB - RPK_RE, C), lambda i: (0, 0)),
        ],
        out_specs=[
            pl.BlockSpec((RPK_KB, C), lambda i: (i, 0)),
            pl.BlockSpec((RPK_KB, C), lambda i: (i, 0)),
        ],
        out_shape=[
            jax.ShapeDtypeStruct((k_chunks, C), jnp.int32),
            jax.ShapeDtypeStruct((k_chunks, C), jnp.int32),
        ],
    )(ei, dummy_src, dummy_dst)


TC_B = 1024


def _tc_linr(x, wr_t, b_l):
    def body(x_ref, wr_ref, b_ref, o_ref):
        o_ref[...] = (
            jnp.dot(x_ref[...], wr_ref[...], preferred_element_type=jnp.float32)
            + b_ref[...]
        )

    return pl.pallas_call(
        body,
        grid=(-(-N // TC_B),),
        in_specs=[
            pl.BlockSpec((TC_B, D), lambda i: (i, 0)),
            pl.BlockSpec((D, D), lambda i: (0, 0)),
            pl.BlockSpec((1, D), lambda i: (0, 0)),
        ],
        out_specs=pl.BlockSpec((TC_B, D), lambda i: (i, 0)),
        out_shape=jax.ShapeDtypeStruct((N, D), jnp.float32),
    )(x, wr_t, b_l)


def _tc_combine(sums, degs, yr, wl_t):
    def body(s_ref, d_ref, yr_ref, wl_ref, o_ref):
        s = s_ref[0] + s_ref[1]
        d = d_ref[0, 0] + d_ref[1, 0]
        mean = s / jnp.maximum(d, 1.0)[:, None]
        o_ref[...] = (
            jnp.dot(mean, wl_ref[...], preferred_element_type=jnp.float32)
            + yr_ref[...]
        )

    return pl.pallas_call(
        body,
        grid=(-(-N // TC_B),),
        in_specs=[
            pl.BlockSpec((NC, TC_B, D), lambda i: (0, i, 0)),
            pl.BlockSpec((NC, 1, TC_B), lambda i: (0, 0, i)),
            pl.BlockSpec((TC_B, D), lambda i: (i, 0)),
            pl.BlockSpec((D, D), lambda i: (0, 0)),
        ],
        out_specs=pl.BlockSpec((TC_B, D), lambda i: (i, 0)),
        out_shape=jax.ShapeDtypeStruct((N, D), jnp.float32),
    )(sums, degs, yr, wl_t)


def kernel(x, edge_index, W_l, b_l, W_r):
    e = edge_index.shape[1]
    k_chunks = (e // (RPK_RE * C)) * RPK_KB
    cpt = k_chunks // NW

    nd = (RPK_KB - RPK_RE) * C
    ar = jnp.arange(nd, dtype=jnp.int32)
    dummy_src = (ar % C).reshape(-1, C)
    dummy_dst = (N + ar % (N_PAD - N)).reshape(-1, C)

    src, dst = _repack(edge_index.astype(jnp.int32), dummy_src, dummy_dst,
                       k_chunks)

    zeros2d = jnp.zeros((C, D), jnp.float32)
    zeros1d = jnp.zeros((DEG_PER_TILE,), jnp.float32)

    sums, degs = _sc_segment_sum(x, src, dst, zeros2d, zeros1d, cpt)
    yr = _tc_linr(x, W_r.T, b_l.reshape(1, D))
    return _tc_combine(sums, degs, yr, W_l.T)


# speedup vs baseline: 1.0304x; 1.0304x over previous
import functools

import jax
import jax.numpy as jnp
from jax import lax
from jax.experimental import pallas as pl
from jax.experimental.pallas import tpu as pltpu
from jax.experimental.pallas import tpu_sc as plsc

N = 10000
D = 128
NC = 2
NS = 16
NW = NC * NS
C = 128

N_PAD = 10112
ROWS_PER_TILE = N_PAD // NS
DEG_PAD = 10240
DEG_PER_TILE = DEG_PAD // NS


def _sc_segment_sum(x_pad, src_chunks, dst_chunks, zeros2d, zeros1d, cpt):
    mesh = plsc.VectorSubcoreMesh(core_axis_name="c", subcore_axis_name="s")

    npc = cpt // 2

    @functools.partial(
        pl.kernel,
        out_type=(
            jax.ShapeDtypeStruct((NC, N_PAD, D), jnp.float32),
            jax.ShapeDtypeStruct((NC, 1, DEG_PAD), jnp.float32),
        ),
        mesh=mesh,
        scratch_types=[
            pltpu.VMEM_SHARED((N_PAD, D), jnp.float32),
            pltpu.VMEM_SHARED((DEG_PAD,), jnp.float32),
            pltpu.VMEM((npc, C), jnp.int32),
            pltpu.VMEM((npc, C), jnp.int32),
            pltpu.VMEM((C, D), jnp.float32),
            pltpu.VMEM((C, D), jnp.float32),
            pltpu.VMEM((C,), jnp.float32),
        ] + [pltpu.SemaphoreType.DMA] * 6,
    )
    def kern(x_hbm, src_hbm, dst_hbm, z2_hbm, z1_hbm, out_sum, out_deg,
             acc_sh, deg_sh, src_v, dst_v, r0, r1, ones_v,
             g0, g1, s0, s1, d0, d1):
        rows = [r0, r1]
        gsems = [g0, g1]
        ssems = [s0, s1]
        dsems = [d0, d1]
        cid = lax.axis_index("c")
        sid = lax.axis_index("s")
        wid = cid * NS + sid

        def fire_gather(j, b):
            pltpu.async_copy(x_hbm.at[src_v.at[j]], rows[b], gsems[b])

        def wait_gather(j, b):
            pltpu.make_async_copy(x_hbm.at[src_v.at[j]], rows[b], gsems[b]).wait()

        def fire_scat(j, b):
            pltpu.async_copy(rows[b], acc_sh.at[dst_v.at[j]], ssems[b], add=True)
            pltpu.async_copy(ones_v, deg_sh.at[dst_v.at[j]], dsems[b], add=True)

        def wait_scat(j, b):
            pltpu.make_async_copy(rows[b], acc_sh.at[dst_v.at[j]], ssems[b]).wait()

        def wait_deg(j, b):
            pltpu.make_async_copy(ones_v, deg_sh.at[dst_v.at[j]], dsems[b]).wait()

        def visit(j, b, wait_prev_scat, fire_g, wait_prev_deg):
            o = 1 - b
            if wait_prev_scat:
                wait_scat(j - 1, o)
            if fire_g:
                fire_gather(j + 1, o)
            if wait_prev_deg:
                wait_deg(j - 2, b)
            wait_gather(j, b)
            fire_scat(j, b)

        pltpu.sync_copy(src_hbm.at[pl.ds(wid * cpt, npc)], src_v)
        pltpu.sync_copy(dst_hbm.at[pl.ds(wid * cpt, npc)], dst_v)
        for i in range(C // 16):
            ones_v[pl.ds(i * 16, 16)] = jnp.full((16,), 1.0, jnp.float32)
        fire_gather(0, 0)
        fire_gather(1, 1)

        zbase = sid * ROWS_PER_TILE
        for r in range(0, ROWS_PER_TILE, C):
            nrows = min(C, ROWS_PER_TILE - r)
            pltpu.sync_copy(z2_hbm.at[pl.ds(0, nrows)],
                            acc_sh.at[pl.ds(zbase + r, nrows)])
        pltpu.sync_copy(z1_hbm, deg_sh.at[pl.ds(sid * DEG_PER_TILE, DEG_PER_TILE)])

        plsc.subcore_barrier()

        for h in range(cpt // npc):
            if h > 0:
                pltpu.sync_copy(src_hbm.at[pl.ds(wid * cpt + h * npc, npc)],
                                src_v)
                pltpu.sync_copy(dst_hbm.at[pl.ds(wid * cpt + h * npc, npc)],
                                dst_v)
                fire_gather(0, 0)
                fire_gather(1, 1)
            visit(0, 0, False, False, False)
            visit(1, 1, True, True, False)

            def body(t, _):
                visit(2 * t, 0, True, True, True)
                visit(2 * t + 1, 1, True, True, True)
                return 0

            lax.fori_loop(1, npc // 2 - 1, body, 0)

            visit(npc - 2, 0, True, True, True)
            visit(npc - 1, 1, True, False, True)
            wait_scat(npc - 1, 1)
            wait_deg(npc - 2, 0)
            wait_deg(npc - 1, 1)

        plsc.subcore_barrier()

        pltpu.sync_copy(acc_sh.at[pl.ds(zbase, ROWS_PER_TILE)],
                        out_sum.at[cid, pl.ds(zbase, ROWS_PER_TILE)])
        pltpu.sync_copy(deg_sh.at[pl.ds(sid * DEG_PER_TILE, DEG_PER_TILE)],
                        out_deg.at[cid, 0,
                                   pl.ds(sid * DEG_PER_TILE, DEG_PER_TILE)])

    return kern(x_pad, src_chunks, dst_chunks, zeros2d, zeros1d)


RPK_KB = 256
RPK_RE = 250


def _repack(ei, dummy_src, dummy_dst, k_chunks):
    eb = RPK_RE * C

    def body(e_ref, dsrc_ref, ddst_ref, osrc_ref, odst_ref):
        osrc_ref[:RPK_RE] = e_ref[0].reshape(RPK_RE, C)
        odst_ref[:RPK_RE] = e_ref[1].reshape(RPK_RE, C)
        osrc_ref[RPK_RE:] = dsrc_ref[...]
        odst_ref[RPK_RE:] = ddst_ref[...]

    return pl.pallas_call(
        body,
        grid=(k_chunks // RPK_KB,),
        in_specs=[
            pl.BlockSpec((2, eb), lambda i: (0, i)),
            pl.BlockSpec((RPK_KB - RPK_RE, C), lambda i: (0, 0)),
            pl.BlockSpec((RPK_KB - RPK_RE, C), lambda i: (0, 0)),
        ],
        out_specs=[
            pl.BlockSpec((RPK_KB, C), lambda i: (i, 0)),
            pl.BlockSpec((RPK_KB, C), lambda i: (i, 0)),
        ],
        out_shape=[
            jax.ShapeDtypeStruct((k_chunks, C), jnp.int32),
            jax.ShapeDtypeStruct((k_chunks, C), jnp.int32),
        ],
    )(ei, dummy_src, dummy_dst)


TC_B = 1024


def _tc_linr(x, wr_t, b_l):
    def body(x_ref, wr_ref, b_ref, o_ref):
        o_ref[...] = (
            jnp.dot(x_ref[...], wr_ref[...], preferred_element_type=jnp.float32)
            + b_ref[...]
        )

    return pl.pallas_call(
        body,
        grid=(-(-N // TC_B),),
        in_specs=[
            pl.BlockSpec((TC_B, D), lambda i: (i, 0)),
            pl.BlockSpec((D, D), lambda i: (0, 0)),
            pl.BlockSpec((1, D), lambda i: (0, 0)),
        ],
        out_specs=pl.BlockSpec((TC_B, D), lambda i: (i, 0)),
        out_shape=jax.ShapeDtypeStruct((N, D), jnp.float32),
    )(x, wr_t, b_l)


def _tc_combine(sums, degs, yr, wl_t):
    def body(s_ref, d_ref, yr_ref, wl_ref, o_ref):
        s = s_ref[0] + s_ref[1]
        d = d_ref[0, 0] + d_ref[1, 0]
        mean = s / jnp.maximum(d, 1.0)[:, None]
        o_ref[...] = (
            jnp.dot(mean, wl_ref[...], preferred_element_type=jnp.float32)
            + yr_ref[...]
        )

    return pl.pallas_call(
        body,
        grid=(-(-N // TC_B),),
        in_specs=[
            pl.BlockSpec((NC, TC_B, D), lambda i: (0, i, 0)),
            pl.BlockSpec((NC, 1, TC_B), lambda i: (0, 0, i)),
            pl.BlockSpec((TC_B, D), lambda i: (i, 0)),
            pl.BlockSpec((D, D), lambda i: (0, 0)),
        ],
        out_specs=pl.BlockSpec((TC_B, D), lambda i: (i, 0)),
        out_shape=jax.ShapeDtypeStruct((N, D), jnp.float32),
    )(sums, degs, yr, wl_t)


def kernel(x, edge_index, W_l, b_l, W_r):
    e = edge_index.shape[1]
    k_chunks = (e // (RPK_RE * C)) * RPK_KB
    cpt = k_chunks // NW

    nd = (RPK_KB - RPK_RE) * C
    ar = jnp.arange(nd, dtype=jnp.int32)
    dummy_src = (ar % C).reshape(-1, C)
    dummy_dst = (N + ar % (N_PAD - N)).reshape(-1, C)

    src, dst = _repack(edge_index.astype(jnp.int32), dummy_src, dummy_dst,
                       k_chunks)

    zeros2d = jnp.zeros((C, D), jnp.float32)
    zeros1d = jnp.zeros((DEG_PER_TILE,), jnp.float32)

    sums, degs = _sc_segment_sum(x, src, dst, zeros2d, zeros1d, cpt)
    yr = _tc_linr(x, W_r.T, b_l.reshape(1, D))
    return _tc_combine(sums, degs, yr, W_l.T)


# speedup vs baseline: 1.0471x; 1.0162x over previous
import functools

import jax
import jax.numpy as jnp
from jax import lax
from jax.experimental import pallas as pl
from jax.experimental.pallas import tpu as pltpu
from jax.experimental.pallas import tpu_sc as plsc

N = 10000
D = 128
NC = 2
NS = 16
NW = NC * NS
C = 128

N_PAD = 10112
ROWS_PER_TILE = N_PAD // NS
DEG_PAD = 10240
DEG_PER_TILE = DEG_PAD // NS


def _sc_segment_sum(x_pad, src_chunks, dst_chunks, zeros2d, zeros1d, cpt):
    mesh = plsc.VectorSubcoreMesh(core_axis_name="c", subcore_axis_name="s")

    npc = cpt // 2

    @functools.partial(
        pl.kernel,
        out_type=(
            jax.ShapeDtypeStruct((NC, N_PAD, D), jnp.float32),
            jax.ShapeDtypeStruct((NC, 1, DEG_PAD), jnp.float32),
        ),
        mesh=mesh,
        scratch_types=[
            pltpu.VMEM_SHARED((N_PAD, D), jnp.float32),
            pltpu.VMEM_SHARED((DEG_PAD,), jnp.float32),
            pltpu.VMEM((npc, C), jnp.int32),
            pltpu.VMEM((npc, C), jnp.int32),
            pltpu.VMEM((C, D), jnp.float32),
            pltpu.VMEM((C, D), jnp.float32),
            pltpu.VMEM((C,), jnp.float32),
        ] + [pltpu.SemaphoreType.DMA] * 6,
    )
    def kern(x_hbm, src_hbm, dst_hbm, z2_hbm, z1_hbm, out_sum, out_deg,
             acc_sh, deg_sh, src_v, dst_v, r0, r1, ones_v,
             g0, g1, s0, s1, d0, d1):
        rows = [r0, r1]
        gsems = [g0, g1]
        ssems = [s0, s1]
        dsems = [d0, d1]
        cid = lax.axis_index("c")
        sid = lax.axis_index("s")
        wid = cid * NS + sid

        def fire_gather(j, b):
            pltpu.async_copy(x_hbm.at[src_v.at[j]], rows[b], gsems[b])

        def wait_gather(j, b):
            pltpu.make_async_copy(x_hbm.at[src_v.at[j]], rows[b], gsems[b]).wait()

        def fire_scat(j, b):
            pltpu.async_copy(rows[b], acc_sh.at[dst_v.at[j]], ssems[b], add=True)
            pltpu.async_copy(ones_v, deg_sh.at[dst_v.at[j]], dsems[b], add=True)

        def wait_scat(j, b):
            pltpu.make_async_copy(rows[b], acc_sh.at[dst_v.at[j]], ssems[b]).wait()

        def wait_deg(j, b):
            pltpu.make_async_copy(ones_v, deg_sh.at[dst_v.at[j]], dsems[b]).wait()

        def visit(j, b, wait_prev_scat, fire_g, wait_prev_deg):
            o = 1 - b
            if wait_prev_scat:
                wait_scat(j - 1, o)
            if fire_g:
                fire_gather(j + 1, o)
            if wait_prev_deg:
                wait_deg(j - 2, b)
            wait_gather(j, b)
            fire_scat(j, b)

        pltpu.sync_copy(src_hbm.at[pl.ds(wid * cpt, npc)], src_v)
        pltpu.sync_copy(dst_hbm.at[pl.ds(wid * cpt, npc)], dst_v)
        for i in range(C // 16):
            ones_v[pl.ds(i * 16, 16)] = jnp.full((16,), 1.0, jnp.float32)
        fire_gather(0, 0)
        fire_gather(1, 1)

        zbase = sid * ROWS_PER_TILE
        for r in range(0, ROWS_PER_TILE, C):
            nrows = min(C, ROWS_PER_TILE - r)
            pltpu.sync_copy(z2_hbm.at[pl.ds(0, nrows)],
                            acc_sh.at[pl.ds(zbase + r, nrows)])
        pltpu.sync_copy(z1_hbm, deg_sh.at[pl.ds(sid * DEG_PER_TILE, DEG_PER_TILE)])

        plsc.subcore_barrier()

        for h in range(cpt // npc):
            if h > 0:
                pltpu.sync_copy(src_hbm.at[pl.ds(wid * cpt + h * npc, npc)],
                                src_v)
                pltpu.sync_copy(dst_hbm.at[pl.ds(wid * cpt + h * npc, npc)],
                                dst_v)
                fire_gather(0, 0)
                fire_gather(1, 1)
            visit(0, 0, False, False, False)
            visit(1, 1, True, True, False)

            def body(t, _):
                visit(2 * t, 0, True, True, True)
                visit(2 * t + 1, 1, True, True, True)
                return 0

            lax.fori_loop(1, npc // 2 - 1, body, 0)

            visit(npc - 2, 0, True, True, True)
            visit(npc - 1, 1, True, False, True)
            wait_scat(npc - 1, 1)
            wait_deg(npc - 2, 0)
            wait_deg(npc - 1, 1)

        plsc.subcore_barrier()

        pltpu.sync_copy(acc_sh.at[pl.ds(zbase, ROWS_PER_TILE)],
                        out_sum.at[cid, pl.ds(zbase, ROWS_PER_TILE)])
        pltpu.sync_copy(deg_sh.at[pl.ds(sid * DEG_PER_TILE, DEG_PER_TILE)],
                        out_deg.at[cid, 0,
                                   pl.ds(sid * DEG_PER_TILE, DEG_PER_TILE)])

    return kern(x_pad, src_chunks, dst_chunks, zeros2d, zeros1d)


RPK_KB = 512
RPK_RE = 500


def _repack(ei, dummy_src, dummy_dst, k_chunks):
    eb = RPK_RE * C

    def body(e_ref, dsrc_ref, ddst_ref, osrc_ref, odst_ref):
        osrc_ref[:RPK_RE] = e_ref[0].reshape(RPK_RE, C)
        odst_ref[:RPK_RE] = e_ref[1].reshape(RPK_RE, C)
        osrc_ref[RPK_RE:] = dsrc_ref[...]
        odst_ref[RPK_RE:] = ddst_ref[...]

    return pl.pallas_call(
        body,
        grid=(k_chunks // RPK_KB,),
        in_specs=[
            pl.BlockSpec((2, eb), lambda i: (0, i)),
            pl.BlockSpec((RPK_KB - RPK_RE, C), lambda i: (0, 0)),
            pl.BlockSpec((RPK_KB - RPK_RE, C), lambda i: (0, 0)),
        ],
        out_specs=[
            pl.BlockSpec((RPK_KB, C), lambda i: (i, 0)),
            pl.BlockSpec((RPK_KB, C), lambda i: (i, 0)),
        ],
        out_shape=[
            jax.ShapeDtypeStruct((k_chunks, C), jnp.int32),
            jax.ShapeDtypeStruct((k_chunks, C), jnp.int32),
        ],
    )(ei, dummy_src, dummy_dst)


TC_B = 1024


def _tc_linr(x, wr_t, b_l):
    def body(x_ref, wr_ref, b_ref, o_ref):
        o_ref[...] = (
            jnp.dot(x_ref[...], wr_ref[...], preferred_element_type=jnp.float32)
            + b_ref[...]
        )

    return pl.pallas_call(
        body,
        grid=(-(-N // TC_B),),
        in_specs=[
            pl.BlockSpec((TC_B, D), lambda i: (i, 0)),
            pl.BlockSpec((D, D), lambda i: (0, 0)),
            pl.BlockSpec((1, D), lambda i: (0, 0)),
        ],
        out_specs=pl.BlockSpec((TC_B, D), lambda i: (i, 0)),
        out_shape=jax.ShapeDtypeStruct((N, D), jnp.float32),
    )(x, wr_t, b_l)


def _tc_combine(sums, degs, yr, wl_t):
    def body(s_ref, d_ref, yr_ref, wl_ref, o_ref):
        s = s_ref[0] + s_ref[1]
        d = d_ref[0, 0] + d_ref[1, 0]
        mean = s / jnp.maximum(d, 1.0)[:, None]
        o_ref[...] = (
            jnp.dot(mean, wl_ref[...], preferred_element_type=jnp.float32)
            + yr_ref[...]
        )

    return pl.pallas_call(
        body,
        grid=(-(-N // TC_B),),
        in_specs=[
            pl.BlockSpec((NC, TC_B, D), lambda i: (0, i, 0)),
            pl.BlockSpec((NC, 1, TC_B), lambda i: (0, 0, i)),
            pl.BlockSpec((TC_B, D), lambda i: (i, 0)),
            pl.BlockSpec((D, D), lambda i: (0, 0)),
        ],
        out_specs=pl.BlockSpec((TC_B, D), lambda i: (i, 0)),
        out_shape=jax.ShapeDtypeStruct((N, D), jnp.float32),
    )(sums, degs, yr, wl_t)


def kernel(x, edge_index, W_l, b_l, W_r):
    e = edge_index.shape[1]
    k_chunks = (e // (RPK_RE * C)) * RPK_KB
    cpt = k_chunks // NW

    nd = (RPK_KB - RPK_RE) * C
    ar = jnp.arange(nd, dtype=jnp.int32)
    dummy_src = (ar % C).reshape(-1, C)
    dummy_dst = (N + ar % (N_PAD - N)).reshape(-1, C)

    src, dst = _repack(edge_index.astype(jnp.int32), dummy_src, dummy_dst,
                       k_chunks)

    zeros2d = jnp.zeros((C, D), jnp.float32)
    zeros1d = jnp.zeros((DEG_PER_TILE,), jnp.float32)

    sums, degs = _sc_segment_sum(x, src, dst, zeros2d, zeros1d, cpt)
    yr = _tc_linr(x, W_r.T, b_l.reshape(1, D))
    return _tc_combine(sums, degs, yr, W_l.T)


# speedup vs baseline: 1.0506x; 1.0033x over previous
import functools

import jax
import jax.numpy as jnp
from jax import lax
from jax.experimental import pallas as pl
from jax.experimental.pallas import tpu as pltpu
from jax.experimental.pallas import tpu_sc as plsc

N = 10000
D = 128
NC = 2
NS = 16
NW = NC * NS
C = 128

N_PAD = 10112
ROWS_PER_TILE = N_PAD // NS
DEG_PAD = 10240
DEG_PER_TILE = DEG_PAD // NS


def _sc_segment_sum(x_pad, src_chunks, dst_chunks, zeros2d, zeros1d, cpt):
    mesh = plsc.VectorSubcoreMesh(core_axis_name="c", subcore_axis_name="s")

    npc = cpt // 2

    @functools.partial(
        pl.kernel,
        out_type=(
            jax.ShapeDtypeStruct((NC, N_PAD, D), jnp.float32),
            jax.ShapeDtypeStruct((NC, 1, DEG_PAD), jnp.float32),
        ),
        mesh=mesh,
        scratch_types=[
            pltpu.VMEM_SHARED((N_PAD, D), jnp.float32),
            pltpu.VMEM_SHARED((DEG_PAD,), jnp.float32),
            pltpu.VMEM((npc, C), jnp.int32),
            pltpu.VMEM((npc, C), jnp.int32),
            pltpu.VMEM((C, D), jnp.float32),
            pltpu.VMEM((C, D), jnp.float32),
            pltpu.VMEM((C,), jnp.float32),
        ] + [pltpu.SemaphoreType.DMA] * 6,
    )
    def kern(x_hbm, src_hbm, dst_hbm, z2_hbm, z1_hbm, out_sum, out_deg,
             acc_sh, deg_sh, src_v, dst_v, r0, r1, ones_v,
             g0, g1, s0, s1, d0, d1):
        rows = [r0, r1]
        gsems = [g0, g1]
        ssems = [s0, s1]
        dsems = [d0, d1]
        cid = lax.axis_index("c")
        sid = lax.axis_index("s")
        wid = cid * NS + sid

        def fire_gather(j, b):
            pltpu.async_copy(x_hbm.at[src_v.at[j]], rows[b], gsems[b])

        def wait_gather(j, b):
            pltpu.make_async_copy(x_hbm.at[src_v.at[j]], rows[b], gsems[b]).wait()

        def fire_scat(j, b):
            pltpu.async_copy(rows[b], acc_sh.at[dst_v.at[j]], ssems[b], add=True)
            pltpu.async_copy(ones_v, deg_sh.at[dst_v.at[j]], dsems[b], add=True)

        def wait_scat(j, b):
            pltpu.make_async_copy(rows[b], acc_sh.at[dst_v.at[j]], ssems[b]).wait()

        def wait_deg(j, b):
            pltpu.make_async_copy(ones_v, deg_sh.at[dst_v.at[j]], dsems[b]).wait()

        def visit(j, b, wait_prev_scat, fire_g, wait_prev_deg):
            o = 1 - b
            if wait_prev_scat:
                wait_scat(j - 1, o)
            if fire_g:
                fire_gather(j + 1, o)
            if wait_prev_deg:
                wait_deg(j - 2, b)
            wait_gather(j, b)
            fire_scat(j, b)

        pltpu.sync_copy(src_hbm.at[pl.ds(wid * cpt, npc)], src_v)
        pltpu.sync_copy(dst_hbm.at[pl.ds(wid * cpt, npc)], dst_v)
        for i in range(C // 16):
            ones_v[pl.ds(i * 16, 16)] = jnp.full((16,), 1.0, jnp.float32)
        fire_gather(0, 0)
        fire_gather(1, 1)

        zbase = sid * ROWS_PER_TILE
        for r in range(0, ROWS_PER_TILE, C):
            nrows = min(C, ROWS_PER_TILE - r)
            pltpu.sync_copy(z2_hbm.at[pl.ds(0, nrows)],
                            acc_sh.at[pl.ds(zbase + r, nrows)])
        pltpu.sync_copy(z1_hbm, deg_sh.at[pl.ds(sid * DEG_PER_TILE, DEG_PER_TILE)])

        plsc.subcore_barrier()

        for h in range(cpt // npc):
            if h > 0:
                pltpu.sync_copy(src_hbm.at[pl.ds(wid * cpt + h * npc, npc)],
                                src_v)
                pltpu.sync_copy(dst_hbm.at[pl.ds(wid * cpt + h * npc, npc)],
                                dst_v)
                fire_gather(0, 0)
                fire_gather(1, 1)
            visit(0, 0, False, False, False)
            visit(1, 1, True, True, False)

            def body(t, _):
                visit(2 * t, 0, True, True, True)
                visit(2 * t + 1, 1, True, True, True)
                return 0

            lax.fori_loop(1, npc // 2 - 1, body, 0)

            visit(npc - 2, 0, True, True, True)
            visit(npc - 1, 1, True, False, True)
            wait_scat(npc - 1, 1)
            wait_deg(npc - 2, 0)
            wait_deg(npc - 1, 1)

        plsc.subcore_barrier()

        pltpu.sync_copy(acc_sh.at[pl.ds(zbase, ROWS_PER_TILE)],
                        out_sum.at[cid, pl.ds(zbase, ROWS_PER_TILE)])
        pltpu.sync_copy(deg_sh.at[pl.ds(sid * DEG_PER_TILE, DEG_PER_TILE)],
                        out_deg.at[cid, 0,
                                   pl.ds(sid * DEG_PER_TILE, DEG_PER_TILE)])

    return kern(x_pad, src_chunks, dst_chunks, zeros2d, zeros1d)


RPK_KB = 640
RPK_RE = 625


def _repack(ei, dummy_src, dummy_dst, k_chunks):
    eb = RPK_RE * C

    def body(e_ref, dsrc_ref, ddst_ref, osrc_ref, odst_ref):
        osrc_ref[:RPK_RE] = e_ref[0].reshape(RPK_RE, C)
        odst_ref[:RPK_RE] = e_ref[1].reshape(RPK_RE, C)
        osrc_ref[RPK_RE:] = dsrc_ref[...]
        odst_ref[RPK_RE:] = ddst_ref[...]

    return pl.pallas_call(
        body,
        grid=(k_chunks // RPK_KB,),
        in_specs=[
            pl.BlockSpec((2, eb), lambda i: (0, i)),
            pl.BlockSpec((RPK_KB - RPK_RE, C), lambda i: (0, 0)),
            pl.BlockSpec((RPK_KB - RPK_RE, C), lambda i: (0, 0)),
        ],
        out_specs=[
            pl.BlockSpec((RPK_KB, C), lambda i: (i, 0)),
            pl.BlockSpec((RPK_KB, C), lambda i: (i, 0)),
        ],
        out_shape=[
            jax.ShapeDtypeStruct((k_chunks, C), jnp.int32),
            jax.ShapeDtypeStruct((k_chunks, C), jnp.int32),
        ],
    )(ei, dummy_src, dummy_dst)


TC_B = 1024


def _tc_linr(x, wr_t, b_l):
    def body(x_ref, wr_ref, b_ref, o_ref):
        o_ref[...] = (
            jnp.dot(x_ref[...], wr_ref[...], preferred_element_type=jnp.float32)
            + b_ref[...]
        )

    return pl.pallas_call(
        body,
        grid=(-(-N // TC_B),),
        in_specs=[
            pl.BlockSpec((TC_B, D), lambda i: (i, 0)),
            pl.BlockSpec((D, D), lambda i: (0, 0)),
            pl.BlockSpec((1, D), lambda i: (0, 0)),
        ],
        out_specs=pl.BlockSpec((TC_B, D), lambda i: (i, 0)),
        out_shape=jax.ShapeDtypeStruct((N, D), jnp.float32),
    )(x, wr_t, b_l)


def _tc_combine(sums, degs, yr, wl_t):
    def body(s_ref, d_ref, yr_ref, wl_ref, o_ref):
        s = s_ref[0] + s_ref[1]
        d = d_ref[0, 0] + d_ref[1, 0]
        mean = s / jnp.maximum(d, 1.0)[:, None]
        o_ref[...] = (
            jnp.dot(mean, wl_ref[...], preferred_element_type=jnp.float32)
            + yr_ref[...]
        )

    return pl.pallas_call(
        body,
        grid=(-(-N // TC_B),),
        in_specs=[
            pl.BlockSpec((NC, TC_B, D), lambda i: (0, i, 0)),
            pl.BlockSpec((NC, 1, TC_B), lambda i: (0, 0, i)),
            pl.BlockSpec((TC_B, D), lambda i: (i, 0)),
            pl.BlockSpec((D, D), lambda i: (0, 0)),
        ],
        out_specs=pl.BlockSpec((TC_B, D), lambda i: (i, 0)),
        out_shape=jax.ShapeDtypeStruct((N, D), jnp.float32),
    )(sums, degs, yr, wl_t)


def kernel(x, edge_index, W_l, b_l, W_r):
    e = edge_index.shape[1]
    k_chunks = (e // (RPK_RE * C)) * RPK_KB
    cpt = k_chunks // NW

    nd = (RPK_KB - RPK_RE) * C
    ar = jnp.arange(nd, dtype=jnp.int32)
    dummy_src = (ar % C).reshape(-1, C)
    dummy_dst = (N + ar % (N_PAD - N)).reshape(-1, C)

    src, dst = _repack(edge_index.astype(jnp.int32), dummy_src, dummy_dst,
                       k_chunks)

    zeros2d = jnp.zeros((C, D), jnp.float32)
    zeros1d = jnp.zeros((DEG_PER_TILE,), jnp.float32)

    sums, degs = _sc_segment_sum(x, src, dst, zeros2d, zeros1d, cpt)
    yr = _tc_linr(x, W_r.T, b_l.reshape(1, D))
    return _tc_combine(sums, degs, yr, W_l.T)
